# baseline (speedup 1.0000x reference)
import functools

import jax
import jax.numpy as jnp
import numpy as np
from jax.experimental import pallas as pl

_ATOMIC_NUMBERS = np.array(
    sorted({1, 2, 4, 5, 6, 7, 8, 9, 12, 14, 15, 16, 17, 18, 20, 22, 30, 31,
            32, 33, 34, 35}),
    dtype=np.int32,
)
_NUM_TYPES = 22
_SCALING = 1.5
_N_NODES = 100000


def _body(nt_ref, atoms_ref, out_ref):
    z = nt_ref[...] + 1
    atoms = atoms_ref[...]
    out_ref[...] = jnp.where(z == atoms, jnp.float32(_SCALING), jnp.float32(0.0))


def kernel(node_type, pos):
    del pos
    n = node_type.shape[0]
    block = 4000
    grid = n // block
    atoms = jnp.asarray(_ATOMIC_NUMBERS).reshape(1, _NUM_TYPES)
    out = pl.pallas_call(
        _body,
        grid=(grid,),
        in_specs=[
            pl.BlockSpec((block, 1), lambda i: (i, 0)),
            pl.BlockSpec((1, _NUM_TYPES), lambda i: (0, 0)),
        ],
        out_specs=pl.BlockSpec((block, _NUM_TYPES), lambda i: (i, 0)),
        out_shape=jax.ShapeDtypeStruct((n, _NUM_TYPES), jnp.float32),
    )(node_type.astype(jnp.int32), atoms)
    return out
